# baseline (speedup 1.0000x reference)
import functools

import jax
import jax.numpy as jnp
from jax import lax
from jax.experimental import pallas as pl
from jax.experimental.pallas import tpu as pltpu
from jax.experimental.pallas import tpu_sc as plsc

_NC = 2
_NS = 16
_NW = _NC * _NS
_C = 128
_NPAD = 10112
_R = 632


def _zero_rows(rows_v, width):
    z16 = jnp.zeros((16,), jnp.float32)
    nsub = width // 16

    def zrow(i, c):
        for j in range(nsub):
            rows_v[i, pl.ds(j * 16, 16)] = z16
        return c
    lax.fori_loop(0, _C, zrow, 0)


def _zero_acc_slice(rows_v, acc_sh, r0):
    rpt = _NPAD // _NS
    nfull = rpt // _C
    rem = rpt - nfull * _C

    def zcopy(k, c):
        pltpu.sync_copy(rows_v, acc_sh.at[pl.ds(r0 + k * _C, _C)])
        return c
    lax.fori_loop(0, nfull, zcopy, 0)
    if rem:
        pltpu.sync_copy(rows_v.at[pl.ds(0, rem)],
                        acc_sh.at[pl.ds(r0 + nfull * _C, rem)])


def _make_agg(width, n_chunks):
    epw = n_chunks * _C
    rpt = _NPAD // _NS

    def body(table_r, src_r, dst_r, out_r, src_v, dst_v, rows_v, acc_sh, sem):
        cid = lax.axis_index("c")
        sid = lax.axis_index("s")
        wid = sid * _NC + cid
        r0 = sid * rpt

        _zero_rows(rows_v, width)
        _zero_acc_slice(rows_v, acc_sh, r0)
        plsc.subcore_barrier()

        base = wid * epw

        def chunk(j, c):
            off = pl.multiple_of(base + j * _C, _C)
            pltpu.sync_copy(src_r.at[pl.ds(off, _C)], src_v)
            pltpu.sync_copy(dst_r.at[pl.ds(off, _C)], dst_v)
            pltpu.async_copy(table_r.at[src_v], rows_v, sem).wait()
            pltpu.sync_copy(rows_v, acc_sh.at[dst_v], add=True)
            return c
        lax.fori_loop(0, n_chunks, chunk, 0)

        plsc.subcore_barrier()
        pltpu.sync_copy(acc_sh.at[pl.ds(r0, rpt)],
                        out_r.at[pl.ds(cid * _NPAD + r0, rpt)])

    mesh = plsc.VectorSubcoreMesh(
        core_axis_name="c", subcore_axis_name="s",
        num_cores=_NC, num_subcores=_NS)
    return pl.kernel(
        body,
        out_type=jax.ShapeDtypeStruct((2 * _NPAD, width), jnp.float32),
        mesh=mesh,
        scratch_types=[
            pltpu.VMEM((_C,), jnp.int32),
            pltpu.VMEM((_C,), jnp.int32),
            pltpu.VMEM((_C, width), jnp.float32),
            pltpu.VMEM_SHARED((_NPAD, width), jnp.float32),
            pltpu.SemaphoreType.DMA,
        ])


def _layer0(feat, deg, w0t, b0r):
    g = _NPAD // _R

    def body(fa, fb, da, db, w, b, o):
        s = fa[...] + fb[...]
        d = da[...] + db[...]
        inv = 1.0 / jnp.maximum(d[:, 0:1], 1.0)
        h = jnp.dot(s * inv, w[...], preferred_element_type=jnp.float32)
        o[...] = jnp.maximum(h + b[...], 0.0)

    return pl.pallas_call(
        body,
        grid=(g,),
        in_specs=[
            pl.BlockSpec((_R, 128), lambda i: (i, 0)),
            pl.BlockSpec((_R, 128), lambda i, g=g: (i + g, 0)),
            pl.BlockSpec((_R, 128), lambda i: (i, 0)),
            pl.BlockSpec((_R, 128), lambda i, g=g: (i + g, 0)),
            pl.BlockSpec((128, 128), lambda i: (0, 0)),
            pl.BlockSpec((1, 128), lambda i: (0, 0)),
        ],
        out_specs=pl.BlockSpec((_R, 128), lambda i: (i, 0)),
        out_shape=jax.ShapeDtypeStruct((_NPAD, 128), jnp.float32),
    )(feat, feat, deg, deg, w0t, b0r)


def _layer1(feat, deg, w1t, b1r, w2at, w2bt):
    g = _NPAD // _R

    def body(fa, fb, da, db, w1, b1, wa, wb, o):
        s = fa[...] + fb[...]
        d = da[...] + db[...]
        inv = 1.0 / jnp.maximum(d[:, 0:1], 1.0)
        t = jnp.dot(s * inv, w1[...], preferred_element_type=jnp.float32)
        t = t + b1[...]
        z = (jnp.dot(t, wa[...], preferred_element_type=jnp.float32)
             + jnp.dot(jnp.maximum(t, 0.0), wb[...],
                       preferred_element_type=jnp.float32))
        o[...] = z

    return pl.pallas_call(
        body,
        grid=(g,),
        in_specs=[
            pl.BlockSpec((_R, 128), lambda i: (i, 0)),
            pl.BlockSpec((_R, 128), lambda i, g=g: (i + g, 0)),
            pl.BlockSpec((_R, 128), lambda i: (i, 0)),
            pl.BlockSpec((_R, 128), lambda i, g=g: (i + g, 0)),
            pl.BlockSpec((128, 128), lambda i: (0, 0)),
            pl.BlockSpec((1, 128), lambda i: (0, 0)),
            pl.BlockSpec((128, 128), lambda i: (0, 0)),
            pl.BlockSpec((128, 128), lambda i: (0, 0)),
        ],
        out_specs=pl.BlockSpec((_R, 128), lambda i: (i, 0)),
        out_shape=jax.ShapeDtypeStruct((_NPAD, 128), jnp.float32),
    )(feat, feat, deg, deg, w1t, b1r, w2at, w2bt)


def _layer2(feat, deg, b2r):
    g = _NPAD // _R

    def body(fa, fb, da, db, b, o):
        s = fa[...] + fb[...]
        d = da[...] + db[...]
        inv = 1.0 / jnp.maximum(d[:, 0:1], 1.0)
        o[...] = s * inv + b[...]

    return pl.pallas_call(
        body,
        grid=(g,),
        in_specs=[
            pl.BlockSpec((_R, 128), lambda i: (i, 0)),
            pl.BlockSpec((_R, 128), lambda i, g=g: (i + g, 0)),
            pl.BlockSpec((_R, 128), lambda i: (i, 0)),
            pl.BlockSpec((_R, 128), lambda i, g=g: (i + g, 0)),
            pl.BlockSpec((1, 128), lambda i: (0, 0)),
        ],
        out_specs=pl.BlockSpec((_R, 128), lambda i: (i, 0)),
        out_shape=jax.ShapeDtypeStruct((_NPAD, 128), jnp.float32),
    )(feat, feat, deg, deg, b2r)


def kernel(x, edge_index, W0, b0, W1, b1, W2, b2):
    n, d_in = x.shape
    e = edge_index.shape[1]
    n_classes = W2.shape[0]

    n_chunks = (e + _NW * _C - 1) // (_NW * _C)
    e_pad = n_chunks * _C * _NW

    src = edge_index[0]
    dst = edge_index[1]
    pad_e = e_pad - e
    srcp = jnp.concatenate([src, jnp.zeros((pad_e,), jnp.int32)])
    dstp = jnp.concatenate([dst, jnp.full((pad_e,), n, jnp.int32)])
    xp = jnp.concatenate([x, jnp.zeros((_NPAD - n, d_in), jnp.float32)])

    b0r = b0.reshape(1, -1)
    b1r = b1.reshape(1, -1)
    w2t = W2.T
    w2at = jnp.zeros((128, 128), jnp.float32).at[:, :n_classes].set(w2t[:128])
    w2bt = jnp.zeros((128, 128), jnp.float32).at[:, :n_classes].set(w2t[128:])
    b2r = jnp.zeros((1, 128), jnp.float32).at[0, :n_classes].set(b2)

    agg128 = _make_agg(128, n_chunks)

    ones_tab = jnp.ones((8, 128), jnp.float32)
    src0 = jnp.zeros_like(srcp)
    deg = agg128(ones_tab, src0, dstp)
    feat0 = agg128(xp, srcp, dstp)
    h0 = _layer0(feat0, deg, W0.T, b0r)
    feat1 = agg128(h0, srcp, dstp)
    z = _layer1(feat1, deg, W1.T, b1r, w2at, w2bt)
    feat2 = agg128(z, srcp, dstp)
    out = _layer2(feat2, deg, b2r)
    return out[:n, :n_classes]


# speedup vs baseline: 8.9000x; 8.9000x over previous
import functools

import jax
import jax.numpy as jnp
from jax import lax
from jax.experimental import pallas as pl
from jax.experimental.pallas import tpu as pltpu
from jax.experimental.pallas import tpu_sc as plsc

_NC = 2
_NS = 16
_NW = _NC * _NS
_C = 128
_NPAD = 10112
_R = 632


def _zero_rows(rows_v, width):
    z16 = jnp.zeros((16,), jnp.float32)
    nsub = width // 16

    def zrow(i, c):
        for j in range(nsub):
            rows_v[i, pl.ds(j * 16, 16)] = z16
        return c
    lax.fori_loop(0, _C, zrow, 0)


def _zero_acc_slice(rows_v, acc_sh, r0):
    rpt = _NPAD // _NS
    nfull = rpt // _C
    rem = rpt - nfull * _C

    def zcopy(k, c):
        pltpu.sync_copy(rows_v, acc_sh.at[pl.ds(r0 + k * _C, _C)])
        return c
    lax.fori_loop(0, nfull, zcopy, 0)
    if rem:
        pltpu.sync_copy(rows_v.at[pl.ds(0, rem)],
                        acc_sh.at[pl.ds(r0 + nfull * _C, rem)])


def _make_agg(width, n_chunks):
    epw = n_chunks * _C
    rpt = _NPAD // _NS

    def body(table_r, src_r, dst_r, out_r, src_v, dst_v, rows_v, acc_sh, sem):
        cid = lax.axis_index("c")
        sid = lax.axis_index("s")
        wid = sid * _NC + cid
        r0 = sid * rpt

        _zero_rows(rows_v, width)
        _zero_acc_slice(rows_v, acc_sh, r0)
        plsc.subcore_barrier()

        base = wid * epw

        def chunk(j, c):
            off = pl.multiple_of(base + j * _C, _C)
            pltpu.sync_copy(src_r.at[pl.ds(off, _C)], src_v)
            pltpu.sync_copy(dst_r.at[pl.ds(off, _C)], dst_v)
            pltpu.async_copy(table_r.at[src_v], rows_v, sem).wait()
            pltpu.sync_copy(rows_v, acc_sh.at[dst_v], add=True)
            return c
        lax.fori_loop(0, n_chunks, chunk, 0)

        plsc.subcore_barrier()
        pltpu.sync_copy(acc_sh.at[pl.ds(r0, rpt)],
                        out_r.at[pl.ds(cid * _NPAD + r0, rpt)])

    mesh = plsc.VectorSubcoreMesh(
        core_axis_name="c", subcore_axis_name="s",
        num_cores=_NC, num_subcores=_NS)
    return pl.kernel(
        body,
        out_type=jax.ShapeDtypeStruct((2 * _NPAD, width), jnp.float32),
        mesh=mesh,
        scratch_types=[
            pltpu.VMEM((_C,), jnp.int32),
            pltpu.VMEM((_C,), jnp.int32),
            pltpu.VMEM((_C, width), jnp.float32),
            pltpu.VMEM_SHARED((_NPAD, width), jnp.float32),
            pltpu.SemaphoreType.DMA,
        ])


def _layer0(feat, deg, w0t, b0r):
    g = _NPAD // _R

    def body(fa, fb, da, db, w, b, o):
        s = fa[...] + fb[...]
        d = da[...] + db[...]
        inv = 1.0 / jnp.maximum(d[:, 0:1], 1.0)
        h = jnp.dot(s * inv, w[...], preferred_element_type=jnp.float32)
        o[...] = jnp.maximum(h + b[...], 0.0)

    return pl.pallas_call(
        body,
        grid=(g,),
        in_specs=[
            pl.BlockSpec((_R, 128), lambda i: (i, 0)),
            pl.BlockSpec((_R, 128), lambda i, g=g: (i + g, 0)),
            pl.BlockSpec((_R, 128), lambda i: (i, 0)),
            pl.BlockSpec((_R, 128), lambda i, g=g: (i + g, 0)),
            pl.BlockSpec((128, 128), lambda i: (0, 0)),
            pl.BlockSpec((1, 128), lambda i: (0, 0)),
        ],
        out_specs=pl.BlockSpec((_R, 128), lambda i: (i, 0)),
        out_shape=jax.ShapeDtypeStruct((_NPAD, 128), jnp.float32),
    )(feat, feat, deg, deg, w0t, b0r)


def _layer1(feat, deg, w1t, b1r, w2at, w2bt):
    g = _NPAD // _R

    def body(fa, fb, da, db, w1, b1, wa, wb, o):
        s = fa[...] + fb[...]
        d = da[...] + db[...]
        inv = 1.0 / jnp.maximum(d[:, 0:1], 1.0)
        t = jnp.dot(s * inv, w1[...], preferred_element_type=jnp.float32)
        t = t + b1[...]
        z = (jnp.dot(t, wa[...], preferred_element_type=jnp.float32)
             + jnp.dot(jnp.maximum(t, 0.0), wb[...],
                       preferred_element_type=jnp.float32))
        o[...] = z

    return pl.pallas_call(
        body,
        grid=(g,),
        in_specs=[
            pl.BlockSpec((_R, 128), lambda i: (i, 0)),
            pl.BlockSpec((_R, 128), lambda i, g=g: (i + g, 0)),
            pl.BlockSpec((_R, 128), lambda i: (i, 0)),
            pl.BlockSpec((_R, 128), lambda i, g=g: (i + g, 0)),
            pl.BlockSpec((128, 128), lambda i: (0, 0)),
            pl.BlockSpec((1, 128), lambda i: (0, 0)),
            pl.BlockSpec((128, 128), lambda i: (0, 0)),
            pl.BlockSpec((128, 128), lambda i: (0, 0)),
        ],
        out_specs=pl.BlockSpec((_R, 128), lambda i: (i, 0)),
        out_shape=jax.ShapeDtypeStruct((_NPAD, 128), jnp.float32),
    )(feat, feat, deg, deg, w1t, b1r, w2at, w2bt)


def _layer2(feat, deg, b2r):
    g = _NPAD // _R

    def body(fa, fb, da, db, b, o):
        s = fa[...] + fb[...]
        d = da[...] + db[...]
        inv = 1.0 / jnp.maximum(d[:, 0:1], 1.0)
        o[...] = s * inv + b[...]

    return pl.pallas_call(
        body,
        grid=(g,),
        in_specs=[
            pl.BlockSpec((_R, 128), lambda i: (i, 0)),
            pl.BlockSpec((_R, 128), lambda i, g=g: (i + g, 0)),
            pl.BlockSpec((_R, 128), lambda i: (i, 0)),
            pl.BlockSpec((_R, 128), lambda i, g=g: (i + g, 0)),
            pl.BlockSpec((1, 128), lambda i: (0, 0)),
        ],
        out_specs=pl.BlockSpec((_R, 128), lambda i: (i, 0)),
        out_shape=jax.ShapeDtypeStruct((_NPAD, 128), jnp.float32),
    )(feat, feat, deg, deg, b2r)


def kernel(x, edge_index, W0, b0, W1, b1, W2, b2):
    n, d_in = x.shape
    e = edge_index.shape[1]
    n_classes = W2.shape[0]

    n_chunks = (e + _NW * _C - 1) // (_NW * _C)
    e_pad = n_chunks * _C * _NW

    src = edge_index[0]
    dst = edge_index[1]
    pad_e = e_pad - e
    srcp = jnp.concatenate([src, jnp.zeros((pad_e,), jnp.int32)])
    dstp = jnp.concatenate([dst, jnp.full((pad_e,), n, jnp.int32)])
    xp = jnp.concatenate([x, jnp.zeros((_NPAD - n, d_in), jnp.float32)])

    b0r = b0.reshape(1, -1)
    b1r = b1.reshape(1, -1)
    w2t = W2.T
    w2at = jnp.zeros((128, 128), jnp.float32).at[:, :n_classes].set(w2t[:128])
    w2bt = jnp.zeros((128, 128), jnp.float32).at[:, :n_classes].set(w2t[128:])
    b2r = jnp.zeros((1, 128), jnp.float32).at[0, :n_classes].set(b2)

    agg128 = _make_agg(128, n_chunks)

    ones_tab = jnp.ones((_NPAD, 128), jnp.float32)
    deg = agg128(ones_tab, srcp, dstp)
    feat0 = agg128(xp, srcp, dstp)
    h0 = _layer0(feat0, deg, W0.T, b0r)
    feat1 = agg128(h0, srcp, dstp)
    z = _layer1(feat1, deg, W1.T, b1r, w2at, w2bt)
    feat2 = agg128(z, srcp, dstp)
    out = _layer2(feat2, deg, b2r)
    return out[:n, :n_classes]
